# baseline (speedup 1.0000x reference)
import functools

import jax
import jax.numpy as jnp
from jax import lax
from jax.experimental import pallas as pl
from jax.experimental.pallas import tpu as pltpu
from jax.experimental.pallas import tpu_sc as plsc

_NW = 32
_CHN = 512
_NBUF = 2
_L = 16


def _sc_gather_tiled(table, indices_t):
    h, n = indices_t.shape
    d = table.shape[1]
    ne0, nr = d // 8, 8
    nn0, nc = _CHN // 128, 128
    nb_n = n // _CHN
    chunks_per_w = nb_n * h // _NW
    assert chunks_per_w % _NBUF == 0 and (nb_n * h) % _NW == 0
    mesh = plsc.VectorSubcoreMesh(core_axis_name="c", subcore_axis_name="s")

    @functools.partial(
        pl.kernel,
        mesh=mesh,
        out_type=jax.ShapeDtypeStruct((h, ne0, n // 128, nr, nc), jnp.float32),
        compiler_params=pltpu.CompilerParams(
            use_tc_tiling_on_sc=False, needs_layout_passes=False),
        scratch_types=[
            pltpu.VMEM((_CHN,), jnp.int32),
            pltpu.VMEM((_CHN,), jnp.int32),
            pltpu.VMEM((_CHN, d), jnp.float32),
            pltpu.VMEM((_CHN, d), jnp.float32),
            pltpu.VMEM((ne0 * nn0 * nr, nc), jnp.float32),
            pltpu.VMEM((ne0 * nn0 * nr, nc), jnp.float32),
            pltpu.SemaphoreType.DMA,
            pltpu.SemaphoreType.DMA,
            pltpu.SemaphoreType.DMA,
            pltpu.SemaphoreType.DMA,
            pltpu.SemaphoreType.DMA,
            pltpu.SemaphoreType.DMA,
        ],
    )
    def k(table_hbm, idx_hbm, out_hbm,
          i0, i1, r0, r1, t0, t1, gi0, gi1, gg0, gg1, gs0, gs1):
        wid = lax.axis_index("s") * 2 + lax.axis_index("c")
        idx_v = (i0, i1)
        rows = (r0, r1)
        tiles = (t0, t1)
        isem = (gi0, gi1)
        gsem = (gg0, gg1)
        ssem = (gs0, gs1)

        def coords(c):
            q = wid * chunks_per_w + c
            return q // nb_n, (q % nb_n) * _CHN

        def idx_desc(c, b):
            hb, nlo = coords(c)
            src = idx_hbm.at[hb, pl.ds(nlo, _CHN)]
            return pltpu.make_async_copy(src, idx_v[b], isem[b])

        def gather_desc(c, b):
            return pltpu.make_async_copy(
                table_hbm.at[idx_v[b]], rows[b], gsem[b])

        def store_descs(c, b):
            hb, nlo = coords(c)
            n0 = nlo // 128
            return [
                pltpu.make_async_copy(
                    tiles[b].at[pl.ds((e0 * nn0 + n1) * nr, nr), :],
                    out_hbm.at[hb, e0, n0 + n1, :, :],
                    ssem[b])
                for e0 in range(ne0) for n1 in range(nn0)
            ]

        def store_start(c, b):
            for dsc in store_descs(c, b):
                dsc.start()

        def store_wait(c, b):
            for dsc in store_descs(c, b):
                dsc.wait()

        def transpose_rows(b):
            @pl.loop(0, ne0 * nn0 * nr * (nc // _L))
            def tbody(i):
                c0 = i % (nc // _L)
                r = (i // (nc // _L)) % nr
                n1 = (i // (nc // _L * nr)) % nn0
                e0 = i // (nc // _L * nr * nn0)
                row = n1 * 128 + c0 * _L + lax.iota(jnp.int32, _L)
                col = lax.iota(jnp.int32, _L) * 0 + (e0 * 8 + r)
                vals = plsc.load_gather(rows[b], [row, col])
                tiles[b][(e0 * nn0 + n1) * nr + r, pl.ds(c0 * _L, _L)] = vals

        for b in range(_NBUF):
            idx_desc(b, b).start()
        for b in range(_NBUF):
            idx_desc(b, b).wait()
            gather_desc(b, b).start()
        for b in range(_NBUF):
            gather_desc(b, b).wait()
            idx_desc(b + _NBUF, b).start()
            transpose_rows(b)
            store_start(b, b)

        @pl.loop(_NBUF, chunks_per_w, step=_NBUF)
        def body(g):
            for b in range(_NBUF):
                c = g + b
                idx_desc(c, b).wait()
                gather_desc(c, b).start()
                gather_desc(c, b).wait()

                @pl.when(c + _NBUF < chunks_per_w)
                def _():
                    idx_desc(c + _NBUF, b).start()

                store_wait(c - _NBUF, b)
                transpose_rows(b)
                store_start(c, b)

        for b in range(_NBUF):
            store_wait(chunks_per_w - _NBUF + b, b)

    return k(table, indices_t)


def kernel(indices, table):
    n, h = indices.shape
    d = table.shape[1]
    t5 = _sc_gather_tiled(table, indices.T)
    return t5.transpose(2, 4, 0, 1, 3).reshape(n, h, d)


# speedup vs baseline: 1.0206x; 1.0206x over previous
import functools

import jax
import jax.numpy as jnp
from jax import lax
from jax.experimental import pallas as pl
from jax.experimental.pallas import tpu as pltpu
from jax.experimental.pallas import tpu_sc as plsc

_NW = 32
_CHN = 512
_NBUF = 2
_L = 16


def _sc_gather_tiled(table, indices_t):
    h, n = indices_t.shape
    d = table.shape[1]
    ne0, nr = d // 8, 8
    nn0, nc = _CHN // 128, 128
    nb_n = n // _CHN
    chunks_per_w = nb_n * h // _NW
    assert chunks_per_w % _NBUF == 0 and (nb_n * h) % _NW == 0
    mesh = plsc.VectorSubcoreMesh(core_axis_name="c", subcore_axis_name="s")

    @functools.partial(
        pl.kernel,
        mesh=mesh,
        out_type=jax.ShapeDtypeStruct((h, ne0, n // 128, nr, nc), jnp.float32),
        compiler_params=pltpu.CompilerParams(
            use_tc_tiling_on_sc=False, needs_layout_passes=False),
        scratch_types=[
            pltpu.VMEM((_CHN,), jnp.int32),
            pltpu.VMEM((_CHN,), jnp.int32),
            pltpu.VMEM((_CHN, d), jnp.float32),
            pltpu.VMEM((_CHN, d), jnp.float32),
            pltpu.VMEM((ne0 * nn0 * nr, nc), jnp.float32),
            pltpu.VMEM((ne0 * nn0 * nr, nc), jnp.float32),
            pltpu.SemaphoreType.DMA,
            pltpu.SemaphoreType.DMA,
            pltpu.SemaphoreType.DMA,
            pltpu.SemaphoreType.DMA,
            pltpu.SemaphoreType.DMA,
            pltpu.SemaphoreType.DMA,
        ],
    )
    def k(table_hbm, idx_hbm, out_hbm,
          i0, i1, r0, r1, t0, t1, gi0, gi1, gg0, gg1, gs0, gs1):
        wid = lax.axis_index("s") * 2 + lax.axis_index("c")
        idx_v = (i0, i1)
        rows = (r0, r1)
        tiles = (t0, t1)
        isem = (gi0, gi1)
        gsem = (gg0, gg1)
        ssem = (gs0, gs1)

        def coords(c):
            q = wid * chunks_per_w + c
            return q // nb_n, (q % nb_n) * _CHN

        def idx_desc(c, b):
            hb, nlo = coords(c)
            src = idx_hbm.at[hb, pl.ds(nlo, _CHN)]
            return pltpu.make_async_copy(src, idx_v[b], isem[b])

        def gather_desc(c, b):
            return pltpu.make_async_copy(
                table_hbm.at[idx_v[b]], rows[b], gsem[b])

        def store_descs(c, b):
            hb, nlo = coords(c)
            n0 = nlo // 128
            return [
                pltpu.make_async_copy(
                    tiles[b].at[pl.ds((e0 * nn0 + n1) * nr, nr), :],
                    out_hbm.at[hb, e0, n0 + n1, :, :],
                    ssem[b])
                for e0 in range(ne0) for n1 in range(nn0)
            ]

        def store_start(c, b):
            for dsc in store_descs(c, b):
                dsc.start()

        def store_wait(c, b):
            for dsc in store_descs(c, b):
                dsc.wait()

        def transpose_rows(b):
            iota = lax.iota(jnp.int32, _L)

            @pl.loop(0, ne0 * nn0)
            def tbody(j):
                e0 = j // nn0
                n1 = j % nn0
                ebase = e0 * 8
                rbase = n1 * 128
                trow = j * nr
                for r in range(nr):
                    col = iota * 0 + (ebase + r)
                    for c0 in range(nc // _L):
                        row = rbase + c0 * _L + iota
                        vals = plsc.load_gather(rows[b], [row, col])
                        tiles[b][trow + r, pl.ds(c0 * _L, _L)] = vals

        for b in range(_NBUF):
            idx_desc(b, b).start()
        for b in range(_NBUF):
            idx_desc(b, b).wait()
            gather_desc(b, b).start()
        for b in range(_NBUF):
            gather_desc(b, b).wait()
            idx_desc(b + _NBUF, b).start()
            transpose_rows(b)
            store_start(b, b)

        @pl.loop(_NBUF, chunks_per_w, step=_NBUF)
        def body(g):
            for b in range(_NBUF):
                c = g + b
                idx_desc(c, b).wait()
                gather_desc(c, b).start()
                gather_desc(c, b).wait()

                @pl.when(c + _NBUF < chunks_per_w)
                def _():
                    idx_desc(c + _NBUF, b).start()

                store_wait(c - _NBUF, b)
                transpose_rows(b)
                store_start(c, b)

        for b in range(_NBUF):
            store_wait(chunks_per_w - _NBUF + b, b)

    return k(table, indices_t)


def kernel(indices, table):
    n, h = indices.shape
    d = table.shape[1]
    t5 = _sc_gather_tiled(table, indices.T)
    return t5.transpose(2, 4, 0, 1, 3).reshape(n, h, d)


# speedup vs baseline: 1.7152x; 1.6806x over previous
import functools

import jax
import jax.numpy as jnp
from jax import lax
from jax.experimental import pallas as pl
from jax.experimental.pallas import tpu as pltpu
from jax.experimental.pallas import tpu_sc as plsc

_NW = 32
_CHN = 512
_NBUF = 2
_L = 16


def _sc_gather_tiled(table, indices_t):
    h, n = indices_t.shape
    d = table.shape[1]
    ne0, nr = d // 8, 8
    nn0, nc = _CHN // 128, 128
    nb_n = n // _CHN
    chunks_per_w = nb_n * h // _NW
    assert chunks_per_w % _NBUF == 0 and (nb_n * h) % _NW == 0
    mesh = plsc.VectorSubcoreMesh(core_axis_name="c", subcore_axis_name="s")

    @functools.partial(
        pl.kernel,
        mesh=mesh,
        out_type=jax.ShapeDtypeStruct((h, ne0, n // 128, nr, nc), jnp.float32),
        compiler_params=pltpu.CompilerParams(
            use_tc_tiling_on_sc=False, needs_layout_passes=False),
        scratch_types=[
            pltpu.VMEM((_CHN,), jnp.int32),
            pltpu.VMEM((_CHN,), jnp.int32),
            pltpu.VMEM((_CHN, d), jnp.float32),
            pltpu.VMEM((_CHN, d), jnp.float32),
            pltpu.VMEM((ne0 * nn0 * nr, nc + 1), jnp.float32),
            pltpu.VMEM((ne0 * nn0 * nr, nc + 1), jnp.float32),
            pltpu.SemaphoreType.DMA,
            pltpu.SemaphoreType.DMA,
            pltpu.SemaphoreType.DMA,
            pltpu.SemaphoreType.DMA,
            pltpu.SemaphoreType.DMA,
            pltpu.SemaphoreType.DMA,
        ],
    )
    def k(table_hbm, idx_hbm, out_hbm,
          i0, i1, r0, r1, t0, t1, gi0, gi1, gg0, gg1, gs0, gs1):
        wid = lax.axis_index("s") * 2 + lax.axis_index("c")
        idx_v = (i0, i1)
        rows = (r0, r1)
        tiles = (t0, t1)
        isem = (gi0, gi1)
        gsem = (gg0, gg1)
        ssem = (gs0, gs1)

        def coords(c):
            q = wid * chunks_per_w + c
            return q // nb_n, (q % nb_n) * _CHN

        def idx_desc(c, b):
            hb, nlo = coords(c)
            src = idx_hbm.at[hb, pl.ds(nlo, _CHN)]
            return pltpu.make_async_copy(src, idx_v[b], isem[b])

        def gather_desc(c, b):
            return pltpu.make_async_copy(
                table_hbm.at[idx_v[b]], rows[b], gsem[b])

        def store_descs(c, b):
            hb, nlo = coords(c)
            n0 = nlo // 128
            return [
                pltpu.make_async_copy(
                    tiles[b].at[pl.ds((e0 * nn0 + n1) * nr, nr), pl.ds(0, nc)],
                    out_hbm.at[hb, e0, n0 + n1, :, :],
                    ssem[b])
                for e0 in range(ne0) for n1 in range(nn0)
            ]

        def store_start(c, b):
            for dsc in store_descs(c, b):
                dsc.start()

        def store_wait(c, b):
            for dsc in store_descs(c, b):
                dsc.wait()

        def transpose_rows(b):
            iota = lax.iota(jnp.int32, _L)
            lane_row = (iota // nr) * (nn0 * nr) + (iota % nr)

            @pl.loop(0, nn0 * (nc // nr))
            def tbody(j0):
                n1 = j0 // (nc // nr)
                cb = j0 % (nc // nr)
                rbase = n1 * 128 + cb * nr
                row0 = lane_row + n1 * nr
                row1 = row0 + 2 * nn0 * nr
                for k in range(nr):
                    v0 = rows[b][rbase + k, pl.ds(0, _L)]
                    v1 = rows[b][rbase + k, pl.ds(_L, _L)]
                    colv = iota * 0 + (cb * nr + k)
                    plsc.store_scatter(tiles[b], [row0, colv], v0)
                    plsc.store_scatter(tiles[b], [row1, colv], v1)

        for b in range(_NBUF):
            idx_desc(b, b).start()
        for b in range(_NBUF):
            idx_desc(b, b).wait()
            gather_desc(b, b).start()
        for b in range(_NBUF):
            gather_desc(b, b).wait()
            idx_desc(b + _NBUF, b).start()
            transpose_rows(b)
            store_start(b, b)

        @pl.loop(_NBUF, chunks_per_w, step=_NBUF)
        def body(g):
            for b in range(_NBUF):
                c = g + b
                idx_desc(c, b).wait()
                gather_desc(c, b).start()
                gather_desc(c, b).wait()

                @pl.when(c + _NBUF < chunks_per_w)
                def _():
                    idx_desc(c + _NBUF, b).start()

                store_wait(c - _NBUF, b)
                transpose_rows(b)
                store_start(c, b)

        for b in range(_NBUF):
            store_wait(chunks_per_w - _NBUF + b, b)

    return k(table, indices_t)


def kernel(indices, table):
    n, h = indices.shape
    d = table.shape[1]
    t5 = _sc_gather_tiled(table, indices.T)
    return t5.transpose(2, 4, 0, 1, 3).reshape(n, h, d)


# speedup vs baseline: 1.9393x; 1.1306x over previous
import functools

import jax
import jax.numpy as jnp
from jax import lax
from jax.experimental import pallas as pl
from jax.experimental.pallas import tpu as pltpu
from jax.experimental.pallas import tpu_sc as plsc

_NW = 32
_CHN = 512
_NBUF = 2
_L = 16


def _sc_gather_tiled(table, indices_t):
    h, n = indices_t.shape
    d = table.shape[1]
    ne0, nr = d // 8, 8
    nn0, nc = _CHN // 128, 128
    nb_n = n // _CHN
    chunks_per_w = nb_n * h // _NW
    assert chunks_per_w % _NBUF == 0 and (nb_n * h) % _NW == 0
    mesh = plsc.VectorSubcoreMesh(core_axis_name="c", subcore_axis_name="s")

    @functools.partial(
        pl.kernel,
        mesh=mesh,
        out_type=jax.ShapeDtypeStruct((h, ne0, n // 128, nr, nc), jnp.float32),
        compiler_params=pltpu.CompilerParams(
            use_tc_tiling_on_sc=False, needs_layout_passes=False),
        scratch_types=[
            pltpu.VMEM((_CHN,), jnp.int32),
            pltpu.VMEM((_CHN,), jnp.int32),
            pltpu.VMEM((_CHN, d), jnp.float32),
            pltpu.VMEM((_CHN, d), jnp.float32),
            pltpu.VMEM((ne0 * nn0 * nr, nc + 1), jnp.float32),
            pltpu.VMEM((ne0 * nn0 * nr, nc + 1), jnp.float32),
            pltpu.SemaphoreType.DMA,
            pltpu.SemaphoreType.DMA,
            pltpu.SemaphoreType.DMA,
            pltpu.SemaphoreType.DMA,
            pltpu.SemaphoreType.DMA,
            pltpu.SemaphoreType.DMA,
        ],
    )
    def k(table_hbm, idx_hbm, out_hbm,
          i0, i1, r0, r1, t0, t1, gi0, gi1, gg0, gg1, gs0, gs1):
        wid = lax.axis_index("s") * 2 + lax.axis_index("c")
        idx_v = (i0, i1)
        rows = (r0, r1)
        tiles = (t0, t1)
        isem = (gi0, gi1)
        gsem = (gg0, gg1)
        ssem = (gs0, gs1)

        def coords(c):
            q = wid * chunks_per_w + c
            return q // nb_n, (q % nb_n) * _CHN

        def idx_desc(c, b):
            hb, nlo = coords(c)
            src = idx_hbm.at[hb, pl.ds(nlo, _CHN)]
            return pltpu.make_async_copy(src, idx_v[b], isem[b])

        def gather_desc(c, b):
            return pltpu.make_async_copy(
                table_hbm.at[idx_v[b]], rows[b], gsem[b])

        def store_descs(c, b):
            hb, nlo = coords(c)
            n0 = nlo // 128
            return [
                pltpu.make_async_copy(
                    tiles[b].at[pl.ds((e0 * nn0 + n1) * nr, nr), pl.ds(0, nc)],
                    out_hbm.at[hb, e0, n0 + n1, :, :],
                    ssem[b])
                for e0 in range(ne0) for n1 in range(nn0)
            ]

        def store_start(c, b):
            for dsc in store_descs(c, b):
                dsc.start()

        def store_wait(c, b):
            for dsc in store_descs(c, b):
                dsc.wait()

        def transpose_rows(b):
            iota = lax.iota(jnp.int32, _L)
            lane_row = (iota // nr) * (nn0 * nr) + (iota % nr)

            @pl.loop(0, nn0 * (nc // nr))
            def tbody(j0):
                n1 = j0 // (nc // nr)
                cb = j0 % (nc // nr)
                rbase = n1 * 128 + cb * nr
                row0 = lane_row + n1 * nr
                row1 = row0 + 2 * nn0 * nr
                for k in range(nr):
                    v0 = rows[b][rbase + k, pl.ds(0, _L)]
                    v1 = rows[b][rbase + k, pl.ds(_L, _L)]
                    colv = iota * 0 + (cb * nr + k)
                    plsc.store_scatter(tiles[b], [row0, colv], v0)
                    plsc.store_scatter(tiles[b], [row1, colv], v1)

        for b in range(_NBUF):
            idx_desc(b, b).start()
        idx_desc(0, 0).wait()
        gather_desc(0, 0).start()

        @pl.loop(0, chunks_per_w, step=_NBUF)
        def body(g):
            for b in range(_NBUF):
                c = g + b
                b1 = 1 - b
                gather_desc(c, b).wait()

                @pl.when(c + 2 < chunks_per_w)
                def _():
                    idx_desc(c + 2, b).start()

                @pl.when(c + 1 < chunks_per_w)
                def _():
                    idx_desc(c + 1, b1).wait()
                    gather_desc(c + 1, b1).start()

                @pl.when(c >= 2)
                def _():
                    store_wait(c - 2, b)

                transpose_rows(b)
                store_start(c, b)

        for b in range(_NBUF):
            store_wait(chunks_per_w - _NBUF + b, b)

    return k(table, indices_t)


def kernel(indices, table):
    n, h = indices.shape
    d = table.shape[1]
    t5 = _sc_gather_tiled(table, indices.T)
    return t5.transpose(2, 4, 0, 1, 3).reshape(n, h, d)
